# baseline (speedup 1.0000x reference)
import functools

import jax
import jax.numpy as jnp
from jax import lax
from jax.experimental import pallas as pl
from jax.experimental.pallas import tpu as pltpu
from jax.experimental.pallas import tpu_sc as plsc


def _tc_body(x_ref, p_ref, o_ref):
    o_ref[...] = x_ref[...] + p_ref[...]


def _kernel_tc(x, pos_table):
    B, S, D = x.shape
    blk = 256
    return pl.pallas_call(
        _tc_body,
        grid=(B, S // blk),
        in_specs=[
            pl.BlockSpec((1, blk, D), lambda b, s: (b, s, 0)),
            pl.BlockSpec((blk, D), lambda b, s: (s, 0)),
        ],
        out_specs=pl.BlockSpec((1, blk, D), lambda b, s: (b, s, 0)),
        out_shape=jax.ShapeDtypeStruct(x.shape, x.dtype),
    )(x, pos_table)


def _kernel_sc(x, pos_table):
    B, S, D = x.shape
    info = plsc.get_sparse_core_info()
    NC, NS, L = info.num_cores, info.num_subcores, info.num_lanes
    NW = NC * NS
    rows_per_w = S // NW
    C = 16
    mesh = plsc.VectorSubcoreMesh(core_axis_name="c", subcore_axis_name="s")

    @functools.partial(
        pl.kernel,
        mesh=mesh,
        out_type=jax.ShapeDtypeStruct((B * S * D,), jnp.float32),
        scratch_types=[
            pltpu.VMEM((C * D,), jnp.float32),
            pltpu.VMEM((C * D,), jnp.float32),
        ],
    )
    def run(x_hbm, pos_hbm, out_hbm, pos_v, x_v):
        wid = lax.axis_index("s") * NC + lax.axis_index("c")
        s0 = wid * rows_per_w
        for c in range(rows_per_w // C):
            pltpu.sync_copy(pos_hbm.at[pl.ds((s0 + c * C) * D, C * D)], pos_v)
            for b in range(B):
                base = (b * S + s0 + c * C) * D
                pltpu.sync_copy(x_hbm.at[pl.ds(base, C * D)], x_v)

                def body(i, _):
                    off = i * L
                    x_v[pl.ds(off, L)] = x_v[pl.ds(off, L)] + pos_v[pl.ds(off, L)]
                    return 0

                lax.fori_loop(0, C * D // L, body, 0)
                pltpu.sync_copy(x_v, out_hbm.at[pl.ds(base, C * D)])

    out = run(x.reshape(-1), pos_table[:S].reshape(-1))
    return out.reshape(B, S, D)


def kernel(x, pos_table):
    return _kernel_sc(x, pos_table)


# speedup vs baseline: 1.5867x; 1.5867x over previous
import functools

import jax
import jax.numpy as jnp
from jax import lax
from jax.experimental import pallas as pl
from jax.experimental.pallas import tpu as pltpu
from jax.experimental.pallas import tpu_sc as plsc


def _kernel_sc(x, pos_table):
    B, S, D = x.shape
    info = plsc.get_sparse_core_info()
    NC, NS, L = info.num_cores, info.num_subcores, info.num_lanes
    NW = NC * NS
    RW = S // NW
    C = 8
    CHUNK = C * D
    NCH = RW // C
    NI = B * NCH
    NB = 4
    LOOKAHEAD = 2

    mesh = plsc.VectorSubcoreMesh(core_axis_name="c", subcore_axis_name="s")

    @functools.partial(
        pl.kernel,
        mesh=mesh,
        out_type=jax.ShapeDtypeStruct((B * S * D,), jnp.float32),
        scratch_types=[pltpu.VMEM((RW * D,), jnp.float32)]
        + [pltpu.VMEM((CHUNK,), jnp.float32) for _ in range(NB)]
        + [pltpu.SemaphoreType.DMA for _ in range(2 * NB + 1)],
    )
    def run(x_hbm, pos_hbm, out_hbm, pos_v, *rest):
        bufs = rest[:NB]
        lsems = rest[NB : 2 * NB]
        ssems = rest[2 * NB : 3 * NB]
        psem = rest[3 * NB]

        wid = lax.axis_index("s") * NC + lax.axis_index("c")
        r0 = wid * RW

        def item_off(i):
            b, c = i // NCH, i % NCH
            return ((b * S + r0 + c * C) * D, c * CHUNK)

        pdesc = pltpu.async_copy(pos_hbm.at[pl.ds(r0 * D, RW * D)], pos_v, psem)

        ldesc = [None] * NB
        sdesc = [None] * NB
        for k in range(LOOKAHEAD):
            hb, _ = item_off(k)
            ldesc[k] = pltpu.async_copy(x_hbm.at[pl.ds(hb, CHUNK)], bufs[k], lsems[k])

        for i in range(NI):
            k = i % NB
            hb, pb = item_off(i)
            ldesc[k].wait()
            if i == 0:
                pdesc.wait()

            buf = bufs[k]

            @plsc.parallel_loop(0, CHUNK, L, unroll=8)
            def add_body(j):
                plsc.addupdate(buf.at[pl.ds(j, L)], pos_v[pl.ds(pb + j, L)])

            sdesc[k] = pltpu.async_copy(buf, out_hbm.at[pl.ds(hb, CHUNK)], ssems[k])

            ni = i + LOOKAHEAD
            if ni < NI:
                nk = ni % NB
                if sdesc[nk] is not None:
                    sdesc[nk].wait()
                nhb, _ = item_off(ni)
                ldesc[nk] = pltpu.async_copy(
                    x_hbm.at[pl.ds(nhb, CHUNK)], bufs[nk], lsems[nk]
                )

        for i in range(max(0, NI - NB), NI):
            sdesc[i % NB].wait()

    out = run(x.reshape(-1), pos_table[:S].reshape(-1))
    return out.reshape(B, S, D)


def kernel(x, pos_table):
    return _kernel_sc(x, pos_table)


# speedup vs baseline: 3.6509x; 2.3010x over previous
import functools

import jax
import jax.numpy as jnp
from jax import lax
from jax.experimental import pallas as pl
from jax.experimental.pallas import tpu as pltpu
from jax.experimental.pallas import tpu_sc as plsc


def _kernel_sc(x, pos_table):
    B, S, D = x.shape
    info = plsc.get_sparse_core_info()
    NC, NS, L = info.num_cores, info.num_subcores, info.num_lanes
    NW = NC * NS
    RW = S // NW
    C = 8
    NCH = RW // C
    NI = B * NCH
    NB = 4
    LOOKAHEAD = 2

    mesh = plsc.VectorSubcoreMesh(core_axis_name="c", subcore_axis_name="s")

    @functools.partial(
        pl.kernel,
        mesh=mesh,
        out_type=jax.ShapeDtypeStruct((B, S, D), jnp.float32),
        compiler_params=pltpu.CompilerParams(use_tc_tiling_on_sc=True),
        scratch_types=[pltpu.VMEM((RW, D), jnp.float32)]
        + [pltpu.VMEM((C, D), jnp.float32) for _ in range(NB)]
        + [pltpu.SemaphoreType.DMA for _ in range(2 * NB + 1)],
    )
    def run(x_hbm, pos_hbm, out_hbm, pos_v, *rest):
        bufs = rest[:NB]
        lsems = rest[NB : 2 * NB]
        ssems = rest[2 * NB : 3 * NB]
        psem = rest[3 * NB]

        wid = lax.axis_index("s") * NC + lax.axis_index("c")
        r0 = wid * RW

        pdesc = pltpu.async_copy(pos_hbm.at[pl.ds(r0, RW), :], pos_v, psem)

        def item_bs(i):
            b, c = i // NCH, i % NCH
            return b, c

        def load(i, k):
            b, c = item_bs(i)
            return pltpu.async_copy(
                x_hbm.at[b, pl.ds(r0 + c * C, C), :], bufs[k], lsems[k]
            )

        ldesc = [None] * NB
        sdesc = [None] * NB
        for k in range(LOOKAHEAD):
            ldesc[k] = load(k, k)

        for i in range(NI):
            k = i % NB
            b, c = item_bs(i)
            ldesc[k].wait()
            if i == 0:
                pdesc.wait()

            buf = bufs[k]
            pbase = c * C

            @plsc.parallel_loop(0, C * D, L, unroll=8)
            def add_body(j):
                r = j // D
                col = j % D
                plsc.addupdate(
                    buf.at[r, pl.ds(col, L)], pos_v[pbase + r, pl.ds(col, L)]
                )

            sdesc[k] = pltpu.async_copy(
                buf, out_hbm.at[b, pl.ds(r0 + c * C, C), :], ssems[k]
            )

            ni = i + LOOKAHEAD
            if ni < NI:
                nk = ni % NB
                if sdesc[nk] is not None:
                    sdesc[nk].wait()
                ldesc[nk] = load(ni, nk)

        for i in range(max(0, NI - NB), NI):
            sdesc[i % NB].wait()

    return run(x, pos_table)


def kernel(x, pos_table):
    return _kernel_sc(x, pos_table)


# speedup vs baseline: 4.0882x; 1.1198x over previous
import functools

import jax
import jax.numpy as jnp
from jax import lax
from jax.experimental import pallas as pl
from jax.experimental.pallas import tpu as pltpu
from jax.experimental.pallas import tpu_sc as plsc


def _kernel_sc(x, pos_table):
    B, S, D = x.shape
    info = plsc.get_sparse_core_info()
    NC, NS, L = info.num_cores, info.num_subcores, info.num_lanes
    NW = NC * NS
    RW = S // NW
    C = 16
    NCH = RW // C
    NI = B * NCH
    NB = 3
    LOOKAHEAD = 2

    mesh = plsc.VectorSubcoreMesh(core_axis_name="c", subcore_axis_name="s")

    @functools.partial(
        pl.kernel,
        mesh=mesh,
        out_type=jax.ShapeDtypeStruct((B, S, D), jnp.float32),
        compiler_params=pltpu.CompilerParams(use_tc_tiling_on_sc=True),
        scratch_types=[pltpu.VMEM((RW, D), jnp.float32)]
        + [pltpu.VMEM((C, D), jnp.float32) for _ in range(NB)]
        + [pltpu.SemaphoreType.DMA for _ in range(2 * NB + 1)],
    )
    def run(x_hbm, pos_hbm, out_hbm, pos_v, *rest):
        bufs = rest[:NB]
        lsems = rest[NB : 2 * NB]
        ssems = rest[2 * NB : 3 * NB]
        psem = rest[3 * NB]

        wid = lax.axis_index("s") * NC + lax.axis_index("c")
        r0 = wid * RW

        pdesc = pltpu.async_copy(pos_hbm.at[pl.ds(r0, RW), :], pos_v, psem)

        def item_bs(i):
            b, c = i // NCH, i % NCH
            return b, c

        def load(i, k):
            b, c = item_bs(i)
            return pltpu.async_copy(
                x_hbm.at[b, pl.ds(r0 + c * C, C), :], bufs[k], lsems[k]
            )

        ldesc = [None] * NB
        sdesc = [None] * NB
        for k in range(LOOKAHEAD):
            ldesc[k] = load(k, k)

        for i in range(NI):
            k = i % NB
            b, c = item_bs(i)
            ldesc[k].wait()
            if i == 0:
                pdesc.wait()

            buf = bufs[k]
            pbase = c * C

            @plsc.parallel_loop(0, C * D, L, unroll=8)
            def add_body(j):
                r = j // D
                col = j % D
                plsc.addupdate(
                    buf.at[r, pl.ds(col, L)], pos_v[pbase + r, pl.ds(col, L)]
                )

            sdesc[k] = pltpu.async_copy(
                buf, out_hbm.at[b, pl.ds(r0 + c * C, C), :], ssems[k]
            )

            ni = i + LOOKAHEAD
            if ni < NI:
                nk = ni % NB
                if sdesc[nk] is not None:
                    sdesc[nk].wait()
                ldesc[nk] = load(ni, nk)

        for i in range(max(0, NI - NB), NI):
            sdesc[i % NB].wait()

    return run(x, pos_table)


def kernel(x, pos_table):
    return _kernel_sc(x, pos_table)


# speedup vs baseline: 4.0952x; 1.0017x over previous
import functools

import jax
import jax.numpy as jnp
from jax import lax
from jax.experimental import pallas as pl
from jax.experimental.pallas import tpu as pltpu
from jax.experimental.pallas import tpu_sc as plsc


def _kernel_sc(x, pos_table):
    B, S, D = x.shape
    info = plsc.get_sparse_core_info()
    NC, NS, L = info.num_cores, info.num_subcores, info.num_lanes
    NW = NC * NS
    RW = S // NW
    C = 16
    NCH = RW // C
    NI = B * NCH
    NB = 3
    LOOKAHEAD = 2

    mesh = plsc.VectorSubcoreMesh(core_axis_name="c", subcore_axis_name="s")

    @functools.partial(
        pl.kernel,
        mesh=mesh,
        out_type=jax.ShapeDtypeStruct((B, S, D), jnp.float32),
        compiler_params=pltpu.CompilerParams(
            use_tc_tiling_on_sc=True, skip_device_barrier=True
        ),
        scratch_types=[pltpu.VMEM((RW, D), jnp.float32)]
        + [pltpu.VMEM((C, D), jnp.float32) for _ in range(NB)]
        + [pltpu.SemaphoreType.DMA for _ in range(2 * NB + 1)],
    )
    def run(x_hbm, pos_hbm, out_hbm, pos_v, *rest):
        bufs = rest[:NB]
        lsems = rest[NB : 2 * NB]
        ssems = rest[2 * NB : 3 * NB]
        psem = rest[3 * NB]

        wid = lax.axis_index("s") * NC + lax.axis_index("c")
        r0 = wid * RW

        pdesc = pltpu.async_copy(pos_hbm.at[pl.ds(r0, RW), :], pos_v, psem)

        def item_bs(i):
            b, c = i // NCH, i % NCH
            return b, c

        def load(i, k):
            b, c = item_bs(i)
            return pltpu.async_copy(
                x_hbm.at[b, pl.ds(r0 + c * C, C), :], bufs[k], lsems[k]
            )

        ldesc = [None] * NB
        sdesc = [None] * NB
        for k in range(LOOKAHEAD):
            ldesc[k] = load(k, k)

        for i in range(NI):
            k = i % NB
            b, c = item_bs(i)
            ldesc[k].wait()
            if i == 0:
                pdesc.wait()

            buf = bufs[k]
            pbase = c * C

            @plsc.parallel_loop(0, C * D, L, unroll=8)
            def add_body(j):
                r = j // D
                col = j % D
                plsc.addupdate(
                    buf.at[r, pl.ds(col, L)], pos_v[pbase + r, pl.ds(col, L)]
                )

            sdesc[k] = pltpu.async_copy(
                buf, out_hbm.at[b, pl.ds(r0 + c * C, C), :], ssems[k]
            )

            ni = i + LOOKAHEAD
            if ni < NI:
                nk = ni % NB
                if sdesc[nk] is not None:
                    sdesc[nk].wait()
                ldesc[nk] = load(ni, nk)

        for i in range(max(0, NI - NB), NI):
            sdesc[i % NB].wait()

    return run(x, pos_table)


def kernel(x, pos_table):
    return _kernel_sc(x, pos_table)
